# baseline (speedup 1.0000x reference)
import functools

import jax
import jax.numpy as jnp
from jax import lax
from jax.experimental import pallas as pl
from jax.experimental.pallas import tpu as pltpu
from jax.experimental.pallas import tpu_sc as plsc

_LANES = 16
_CHUNK = 128


@functools.cache
def _build(B, D, n_rows):
    info = plsc.get_sparse_core_info()
    NC, NS = info.num_cores, info.num_subcores
    NW = NC * NS
    b_per_w = B // NW
    n_chunks = b_per_w // _CHUNK

    mesh = plsc.VectorSubcoreMesh(core_axis_name="c", subcore_axis_name="s")

    @functools.partial(
        pl.kernel,
        mesh=mesh,
        compiler_params=pltpu.CompilerParams(use_tc_tiling_on_sc=False),
        out_type=jax.ShapeDtypeStruct((D, B), jnp.float32),
        scratch_types=[
            pltpu.VMEM((n_chunks, _CHUNK), jnp.int32),
            pltpu.VMEM((n_chunks, _CHUNK), jnp.int32),
            pltpu.VMEM((D, b_per_w), jnp.float32),
            pltpu.VMEM((D, b_per_w), jnp.float32),
            pltpu.SemaphoreType.DMA,
        ],
    )
    def gmf(uid_hbm, iid_hbm, utt_hbm, itt_hbm, out_hbm,
            uidx, iidx, urows, irows, sem):
        wid = lax.axis_index("s") * NC + lax.axis_index("c")
        base = wid * b_per_w
        for c in range(n_chunks):
            pltpu.sync_copy(uid_hbm.at[pl.ds(base + c * _CHUNK, _CHUNK)],
                            uidx.at[c])
            pltpu.sync_copy(iid_hbm.at[pl.ds(base + c * _CHUNK, _CHUNK)],
                            iidx.at[c])
        copies = []
        for d in range(D):
            for c in range(n_chunks):
                copies.append(pltpu.async_copy(
                    utt_hbm.at[d].at[uidx.at[c]],
                    urows.at[d, pl.ds(c * _CHUNK, _CHUNK)], sem))
                copies.append(pltpu.async_copy(
                    itt_hbm.at[d].at[iidx.at[c]],
                    irows.at[d, pl.ds(c * _CHUNK, _CHUNK)], sem))
        for cp in copies:
            cp.wait()

        def mul_vec(k, carry):
            for d in range(D):
                u = urows[d, pl.ds(k * _LANES, _LANES)]
                v = irows[d, pl.ds(k * _LANES, _LANES)]
                urows[d, pl.ds(k * _LANES, _LANES)] = u * v
            return carry

        lax.fori_loop(0, b_per_w // _LANES, mul_vec, 0)
        pltpu.sync_copy(urows, out_hbm.at[:, pl.ds(base, b_per_w)])

    return gmf


def kernel(user_ids, item_ids, user_table, item_table):
    B, = user_ids.shape
    D = user_table.shape[1]
    gmf = _build(B, D, user_table.shape[0])
    out_t = gmf(user_ids.astype(jnp.int32), item_ids.astype(jnp.int32),
                user_table.T, item_table.T)
    return out_t.T


# speedup vs baseline: 5.5212x; 5.5212x over previous
import functools

import jax
import jax.numpy as jnp
from jax import lax
from jax.experimental import pallas as pl
from jax.experimental.pallas import tpu as pltpu
from jax.experimental.pallas import tpu_sc as plsc

_LANES = 16
_CHUNK = 128


@functools.cache
def _build(B, D, n_rows4):
    info = plsc.get_sparse_core_info()
    NC, NS = info.num_cores, info.num_subcores
    NW = NC * NS
    b_per_w = B // NW
    n_chunks = b_per_w // _CHUNK
    n_groups = _CHUNK // _LANES

    mesh = plsc.VectorSubcoreMesh(core_axis_name="c", subcore_axis_name="s")

    @functools.partial(
        pl.kernel,
        mesh=mesh,
        compiler_params=pltpu.CompilerParams(use_tc_tiling_on_sc=True,
                                             needs_layout_passes=False),
        out_type=jax.ShapeDtypeStruct((D, B), jnp.float32),
        scratch_types=[
            pltpu.VMEM((n_chunks, _CHUNK), jnp.int32),
            pltpu.VMEM((n_chunks, _CHUNK), jnp.int32),
            pltpu.VMEM((n_chunks, _CHUNK), jnp.int32),
            pltpu.VMEM((n_chunks, _CHUNK), jnp.int32),
            pltpu.VMEM((2, _CHUNK, 128), jnp.float32),
            pltpu.VMEM((2, _CHUNK, 128), jnp.float32),
            pltpu.VMEM((D, b_per_w), jnp.float32),
            pltpu.SemaphoreType.DMA,
        ],
    )
    def gmf(urow_hbm, irow_hbm, ucol_hbm, icol_hbm, ut4_hbm, it4_hbm, out_hbm,
            uidx, iidx, ucol, icol, ubuf, ibuf, prod, sem):
        wid = lax.axis_index("s") * NC + lax.axis_index("c")
        base = wid * b_per_w
        for c in range(n_chunks):
            sl = pl.ds(base + c * _CHUNK, _CHUNK)
            pltpu.sync_copy(urow_hbm.at[sl], uidx.at[c])
            pltpu.sync_copy(irow_hbm.at[sl], iidx.at[c])
            pltpu.sync_copy(ucol_hbm.at[sl], ucol.at[c])
            pltpu.sync_copy(icol_hbm.at[sl], icol.at[c])

        def fire(c):
            slot = c % 2
            cp_u = pltpu.async_copy(ut4_hbm.at[uidx.at[c]], ubuf.at[slot], sem)
            cp_i = pltpu.async_copy(it4_hbm.at[iidx.at[c]], ibuf.at[slot], sem)
            return cp_u, cp_i

        pending = fire(0)
        for c in range(n_chunks):
            pending[0].wait()
            pending[1].wait()
            if c + 1 < n_chunks:
                pending = fire(c + 1)
            slot = c % 2

            def group(g, carry):
                rows = jax.lax.broadcasted_iota(jnp.int32, (_LANES,), 0) \
                    + g * _LANES
                cu = ucol[c, pl.ds(g * _LANES, _LANES)]
                ci = icol[c, pl.ds(g * _LANES, _LANES)]
                for d in range(D):
                    u = plsc.load_gather(ubuf.at[slot], [rows, cu + d])
                    v = plsc.load_gather(ibuf.at[slot], [rows, ci + d])
                    prod[d, pl.ds(c * _CHUNK + g * _LANES, _LANES)] = u * v
                return carry

            lax.fori_loop(0, n_groups, group, 0)

        pltpu.sync_copy(prod, out_hbm.at[:, pl.ds(base, b_per_w)])

    return gmf


def kernel(user_ids, item_ids, user_table, item_table):
    B, = user_ids.shape
    n_rows, D = user_table.shape
    uid = user_ids.astype(jnp.int32)
    iid = item_ids.astype(jnp.int32)
    gmf = _build(B, D, n_rows // 4)
    out_t = gmf(uid >> 2, iid >> 2, (uid & 3) * D, (iid & 3) * D,
                user_table.reshape(n_rows // 4, 4 * D),
                item_table.reshape(n_rows // 4, 4 * D))
    return out_t.T


# speedup vs baseline: 21.6152x; 3.9150x over previous
import functools

import jax
import jax.numpy as jnp
from jax import lax
from jax.experimental import pallas as pl
from jax.experimental.pallas import tpu as pltpu
from jax.experimental.pallas import tpu_sc as plsc

_LANES = 16
_NB = 8


@functools.cache
def _build(B, D, n_rows):
    info = plsc.get_sparse_core_info()
    NC, NS = info.num_cores, info.num_subcores
    NW = NC * NS
    assert B % (NW * _NB) == 0 and D % _LANES == 0
    b_per_w = B // NW
    n_groups = b_per_w // _NB
    n_full_tc = n_rows // 128
    tail_start = n_full_tc * 128
    tail_n = n_rows - tail_start
    max_tc = n_full_tc - 1

    mesh = plsc.VectorSubcoreMesh(core_axis_name="c", subcore_axis_name="s")

    @functools.partial(
        pl.kernel,
        mesh=mesh,
        compiler_params=pltpu.CompilerParams(use_tc_tiling_on_sc=True,
                                             needs_layout_passes=False),
        out_type=jax.ShapeDtypeStruct((D, B), jnp.float32),
        scratch_types=[
            pltpu.VMEM((b_per_w,), jnp.int32),
            pltpu.VMEM((b_per_w,), jnp.int32),
            pltpu.VMEM((_NB, D, 128), jnp.float32),
            pltpu.VMEM((_NB, D, 128), jnp.float32),
            pltpu.VMEM((max(tail_n, 1), 128), jnp.float32),
            pltpu.VMEM((max(tail_n, 1), 128), jnp.float32),
            pltpu.VMEM((D, b_per_w), jnp.float32),
            pltpu.SemaphoreType.DMA((_NB,)),
            pltpu.SemaphoreType.DMA((_NB,)),
            pltpu.SemaphoreType.DMA,
        ],
    )
    def gmf(uid_hbm, iid_hbm, utt_hbm, itt_hbm, utail_hbm, itail_hbm, out_hbm,
            uids_v, iids_v, ublk, iblk, utailv, itailv, prod,
            usem, isem, sem):
        wid = lax.axis_index("s") * NC + lax.axis_index("c")
        base = wid * b_per_w
        pltpu.sync_copy(uid_hbm.at[pl.ds(base, b_per_w)], uids_v)
        pltpu.sync_copy(iid_hbm.at[pl.ds(base, b_per_w)], iids_v)
        if tail_n:
            pltpu.sync_copy(utail_hbm, utailv)
            pltpu.sync_copy(itail_hbm, itailv)

        iota = jax.lax.broadcasted_iota(jnp.int32, (_LANES,), 0)

        def extract(ids_v, k):
            vec = ids_v[pl.ds((k >> 4) * _LANES, _LANES)]
            return jnp.max(jnp.where(iota == (k & 15), vec, 0))

        def fire(slot, r, ri):
            tcu = jnp.minimum(r >> 7, max_tc)
            tci = jnp.minimum(ri >> 7, max_tc)
            pltpu.async_copy(utt_hbm.at[:, pl.ds(tcu * 128, 128)],
                             ublk.at[slot], usem.at[slot])
            pltpu.async_copy(itt_hbm.at[:, pl.ds(tci * 128, 128)],
                             iblk.at[slot], isem.at[slot])

        def drain(slot):
            pltpu.make_async_copy(utt_hbm.at[:, pl.ds(0, 128)],
                                  ublk.at[slot], usem.at[slot]).wait()
            pltpu.make_async_copy(utt_hbm.at[:, pl.ds(0, 128)],
                                  iblk.at[slot], isem.at[slot]).wait()

        def take(blk, slot, tailv, r):
            col = jnp.full((_LANES,), r & 127, jnp.int32)
            lo = plsc.load_gather(blk.at[slot], [iota, col])
            hi = plsc.load_gather(blk.at[slot], [iota + _LANES, col])
            if tail_n:
                rt = jnp.maximum(r - tail_start, 0)
                is_tail = r >= tail_start
                lo = jnp.where(is_tail, tailv[rt, pl.ds(0, _LANES)], lo)
                hi = jnp.where(is_tail, tailv[rt, pl.ds(_LANES, _LANES)], hi)
            return lo, hi

        carry0 = []
        for j in range(_NB):
            r = extract(uids_v, j)
            ri = extract(iids_v, j)
            fire(j, r, ri)
            carry0 += [r, ri]

        def group(g, carry):
            new = []
            for j in range(_NB):
                k = g * _NB + j
                r, ri = carry[2 * j], carry[2 * j + 1]
                drain(j)
                u_lo, u_hi = take(ublk, j, utailv, r)
                v_lo, v_hi = take(iblk, j, itailv, ri)
                colk = jnp.full((_LANES,), k, jnp.int32)
                plsc.store_scatter(prod, [iota, colk], u_lo * v_lo)
                plsc.store_scatter(prod, [iota + _LANES, colk], u_hi * v_hi)
                kn = jnp.minimum(k + _NB, b_per_w - 1)
                rn = extract(uids_v, kn)
                rin = extract(iids_v, kn)
                fire(j, rn, rin)
                new += [rn, rin]
            return tuple(new)

        lax.fori_loop(0, n_groups, group, tuple(carry0))
        for j in range(_NB):
            drain(j)

        pltpu.sync_copy(prod, out_hbm.at[:, pl.ds(base, b_per_w)])

    return gmf


def kernel(user_ids, item_ids, user_table, item_table):
    B, = user_ids.shape
    n_rows, D = user_table.shape
    tail_start = (n_rows // 128) * 128
    pad = ((0, 0), (0, 128 - D))
    gmf = _build(B, D, n_rows)
    out_t = gmf(user_ids.astype(jnp.int32), item_ids.astype(jnp.int32),
                user_table.T, item_table.T,
                jnp.pad(user_table[tail_start:], pad),
                jnp.pad(item_table[tail_start:], pad))
    return out_t.T


# speedup vs baseline: 21.6292x; 1.0006x over previous
import functools

import jax
import jax.numpy as jnp
from jax import lax
from jax.experimental import pallas as pl
from jax.experimental.pallas import tpu as pltpu
from jax.experimental.pallas import tpu_sc as plsc

_LANES = 16
_NB = 8


@functools.cache
def _build(B, D, n_rows):
    info = plsc.get_sparse_core_info()
    NC, NS = info.num_cores, info.num_subcores
    NW = NC * NS
    assert B % (NW * _NB) == 0 and D % _LANES == 0
    b_per_w = B // NW
    n_groups = b_per_w // _NB
    n_full_tc = n_rows // 128
    tail_start = n_full_tc * 128
    tail_n = n_rows - tail_start
    max_tc = n_full_tc - 1

    mesh = plsc.VectorSubcoreMesh(core_axis_name="c", subcore_axis_name="s")

    @functools.partial(
        pl.kernel,
        mesh=mesh,
        compiler_params=pltpu.CompilerParams(use_tc_tiling_on_sc=True,
                                             needs_layout_passes=False),
        out_type=jax.ShapeDtypeStruct((D, B), jnp.float32),
        scratch_types=[
            pltpu.VMEM((b_per_w,), jnp.int32),
            pltpu.VMEM((b_per_w,), jnp.int32),
            pltpu.VMEM((_NB, D, 128), jnp.float32),
            pltpu.VMEM((_NB, D, 128), jnp.float32),
            pltpu.VMEM((max(tail_n, 1), 128), jnp.float32),
            pltpu.VMEM((max(tail_n, 1), 128), jnp.float32),
            pltpu.VMEM((D, b_per_w), jnp.float32),
            pltpu.SemaphoreType.DMA((_NB,)),
            pltpu.SemaphoreType.DMA((_NB,)),
            pltpu.SemaphoreType.DMA,
        ],
    )
    def gmf(uid_hbm, iid_hbm, utt_hbm, itt_hbm, utail_hbm, itail_hbm, out_hbm,
            uids_v, iids_v, ublk, iblk, utailv, itailv, prod,
            usem, isem, sem):
        wid = lax.axis_index("s") * NC + lax.axis_index("c")
        base = wid * b_per_w
        pltpu.sync_copy(uid_hbm.at[pl.ds(base, b_per_w)], uids_v)
        pltpu.sync_copy(iid_hbm.at[pl.ds(base, b_per_w)], iids_v)
        if tail_n:
            pltpu.sync_copy(utail_hbm, utailv)
            pltpu.sync_copy(itail_hbm, itailv)

        iota = jax.lax.broadcasted_iota(jnp.int32, (_LANES,), 0)

        def extract(ids_v, k):
            vec = ids_v[pl.ds((k >> 4) * _LANES, _LANES)]
            return jnp.max(jnp.where(iota == (k & 15), vec, 0))

        def fire(slot, r, ri):
            tcu = jnp.minimum(r >> 7, max_tc)
            tci = jnp.minimum(ri >> 7, max_tc)
            pltpu.async_copy(utt_hbm.at[:, pl.ds(tcu * 128, 128)],
                             ublk.at[slot], usem.at[slot])
            pltpu.async_copy(itt_hbm.at[:, pl.ds(tci * 128, 128)],
                             iblk.at[slot], isem.at[slot])

        def drain(slot):
            pltpu.make_async_copy(utt_hbm.at[:, pl.ds(0, 128)],
                                  ublk.at[slot], usem.at[slot]).wait()
            pltpu.make_async_copy(utt_hbm.at[:, pl.ds(0, 128)],
                                  iblk.at[slot], isem.at[slot]).wait()

        def take(blk, slot, tailv, r):
            col = jnp.full((_LANES,), r & 127, jnp.int32)
            lo = plsc.load_gather(blk.at[slot], [iota, col])
            hi = plsc.load_gather(blk.at[slot], [iota + _LANES, col])
            if tail_n:
                rt = jnp.full((_LANES,), jnp.maximum(r - tail_start, 0),
                              jnp.int32)
                is_tail = r >= tail_start
                t_lo = plsc.load_gather(tailv, [rt, iota])
                t_hi = plsc.load_gather(tailv, [rt, iota + _LANES])
                lo = jnp.where(is_tail, t_lo, lo)
                hi = jnp.where(is_tail, t_hi, hi)
            return lo, hi

        carry0 = []
        for j in range(_NB):
            r = extract(uids_v, j)
            ri = extract(iids_v, j)
            fire(j, r, ri)
            carry0 += [r, ri]

        def group(g, carry):
            new = []
            for j in range(_NB):
                k = g * _NB + j
                r, ri = carry[2 * j], carry[2 * j + 1]
                drain(j)
                u_lo, u_hi = take(ublk, j, utailv, r)
                v_lo, v_hi = take(iblk, j, itailv, ri)
                colk = jnp.full((_LANES,), k, jnp.int32)
                plsc.store_scatter(prod, [iota, colk], u_lo * v_lo)
                plsc.store_scatter(prod, [iota + _LANES, colk], u_hi * v_hi)
                kn = jnp.minimum(k + _NB, b_per_w - 1)
                rn = extract(uids_v, kn)
                rin = extract(iids_v, kn)
                fire(j, rn, rin)
                new += [rn, rin]
            return tuple(new)

        lax.fori_loop(0, n_groups, group, tuple(carry0))
        for j in range(_NB):
            drain(j)

        pltpu.sync_copy(prod, out_hbm.at[:, pl.ds(base, b_per_w)])

    return gmf


def kernel(user_ids, item_ids, user_table, item_table):
    B, = user_ids.shape
    n_rows, D = user_table.shape
    tail_start = (n_rows // 128) * 128
    pad = ((0, 0), (0, 128 - D))
    gmf = _build(B, D, n_rows)
    out_t = gmf(user_ids.astype(jnp.int32), item_ids.astype(jnp.int32),
                user_table.T, item_table.T,
                jnp.pad(user_table[tail_start:], pad),
                jnp.pad(item_table[tail_start:], pad))
    return out_t.T
